# Initial kernel scaffold; baseline (speedup 1.0000x reference)
import jax
import jax.numpy as jnp
from jax.experimental import pallas as pl


def kernel(input_ids, weight):
    raise NotImplementedError("write your pallas kernel here")


# speedup vs baseline: 4.0942x; 4.0942x over previous
import functools

import jax
import jax.numpy as jnp
from jax import lax
from jax.experimental import pallas as pl
from jax.experimental.pallas import tpu as pltpu
from jax.experimental.pallas import tpu_sc as plsc

B = 4096
H = 50
V = 100000
D = 64
N = B * H

NC = 2
NS = 16
NW = NC * NS
PER_W = N // NW
CH = 128
NCH = PER_W // CH

_mesh = plsc.VectorSubcoreMesh(core_axis_name="c", subcore_axis_name="s")


@functools.partial(
    pl.kernel,
    mesh=_mesh,
    out_type=jax.ShapeDtypeStruct((N, D), jnp.float32),
    scratch_types=[
        pltpu.VMEM((NCH, CH), jnp.int32),
        pltpu.VMEM((CH, D), jnp.float32),
        pltpu.SemaphoreType.DMA,
    ],
    compiler_params=pltpu.CompilerParams(use_tc_tiling_on_sc=False),
)
def _emb_lookup(idx_hbm, w_hbm, out_hbm, idx_v, rows_v, gsem):
    wid = lax.axis_index("s") * NC + lax.axis_index("c")
    base = wid * PER_W

    pltpu.sync_copy(idx_hbm.at[wid], idx_v)

    def chunk_body(g, carry):
        pltpu.async_copy(w_hbm.at[idx_v.at[g]], rows_v, gsem).wait()
        pltpu.sync_copy(rows_v, out_hbm.at[pl.ds(base + g * CH, CH)])
        return carry

    lax.fori_loop(0, NCH, chunk_body, 0)


def kernel(input_ids, weight):
    ids = input_ids.reshape(NW, NCH, CH).astype(jnp.int32)
    out = _emb_lookup(ids, weight)
    return out.reshape(B, H, D)


# speedup vs baseline: 4.6907x; 1.1457x over previous
import functools

import jax
import jax.numpy as jnp
from jax import lax
from jax.experimental import pallas as pl
from jax.experimental.pallas import tpu as pltpu
from jax.experimental.pallas import tpu_sc as plsc

B = 4096
H = 50
V = 100000
D = 64
N = B * H

NC = 2
NS = 16
NW = NC * NS
PER_W = N // NW
CH = 128
NCH = PER_W // CH
NBUF = 5
LA = 3
T = NCH // NBUF

_mesh = plsc.VectorSubcoreMesh(core_axis_name="c", subcore_axis_name="s")


@functools.partial(
    pl.kernel,
    mesh=_mesh,
    out_type=jax.ShapeDtypeStruct((N, D), jnp.float32),
    scratch_types=[
        pltpu.VMEM((NCH, CH), jnp.int32),
        pltpu.VMEM((NBUF, CH, D), jnp.float32),
        pltpu.SemaphoreType.DMA,
        pltpu.SemaphoreType.DMA,
    ],
    compiler_params=pltpu.CompilerParams(use_tc_tiling_on_sc=False),
)
def _emb_lookup(idx_hbm, w_hbm, out_hbm, idx_v, rows_v, gsem, ssem):
    wid = lax.axis_index("s") * NC + lax.axis_index("c")
    base = wid * PER_W

    pltpu.sync_copy(idx_hbm.at[wid], idx_v)

    def issue_gather(g, slot):
        pltpu.async_copy(w_hbm.at[idx_v.at[g]], rows_v.at[slot], gsem)

    def wait_gather(g, slot):
        pltpu.make_async_copy(w_hbm.at[idx_v.at[g]], rows_v.at[slot], gsem).wait()

    def issue_store(g, slot):
        pltpu.async_copy(rows_v.at[slot], out_hbm.at[pl.ds(base + g * CH, CH)], ssem)

    def wait_one_store():
        pltpu.make_async_copy(
            rows_v.at[0], out_hbm.at[pl.ds(base, CH)], ssem
        ).wait()

    def step(g, b, store_wait, issue):
        if store_wait:
            wait_one_store()
        if issue:
            issue_gather(g + LA, (b + LA) % NBUF)
        wait_gather(g, b)
        issue_store(g, b)

    for g in range(LA):
        issue_gather(g, g)

    for b in range(NBUF):
        step(b, b, store_wait=(b >= NBUF - LA), issue=True)

    def outer(t, carry):
        for b in range(NBUF):
            step(t * NBUF + b, b, store_wait=True, issue=True)
        return carry

    lax.fori_loop(1, T - 1, outer, 0)

    for b in range(NBUF):
        g = (T - 1) * NBUF + b
        step(g, b, store_wait=True, issue=(g + LA < NCH))

    for _ in range(NBUF - LA):
        wait_one_store()


def kernel(input_ids, weight):
    ids = input_ids.reshape(NW, NCH, CH).astype(jnp.int32)
    out = _emb_lookup(ids, weight)
    return out.reshape(B, H, D)
